# Initial kernel scaffold; baseline (speedup 1.0000x reference)
import jax
import jax.numpy as jnp
from jax.experimental import pallas as pl


def kernel(x_user, x_item, ei_rates, ei_rev, Wl_rates, bl_rates, Wr_rates, Wl_rev, bl_rev, Wr_rev, g_user, b_user, g_item, b_item):
    raise NotImplementedError("write your pallas kernel here")


# speedup vs baseline: 1.9211x; 1.9211x over previous
import functools

import jax
import jax.numpy as jnp
from jax import lax
from jax.experimental import pallas as pl
from jax.experimental.pallas import tpu as pltpu
from jax.experimental.pallas import tpu_sc as plsc

N = 50000
D = 128

NC = 2
NS = 16
LANES = 16

B = 128
R = 12800
PAD = 128
NCHUNK = -(-N // R)
OUTR = NCHUNK * R


EB = 896
DEGR = 50176


def _sc_segment_sum(xcat, src_cat, dst_cat, ep):
    te = ep // NS
    nstage = te // EB
    zrows = (R + PAD) // NS
    orows = R // NS

    zeros_acc = jnp.zeros((R + PAD, D), jnp.float32)

    mesh = plsc.VectorSubcoreMesh(core_axis_name="c", subcore_axis_name="s",
                                  num_cores=NC, num_subcores=NS)

    @functools.partial(
        pl.kernel,
        mesh=mesh,
        out_type=jax.ShapeDtypeStruct((NC * OUTR, D), jnp.float32),
        scratch_types=[
            pltpu.VMEM((EB,), jnp.int32),
            pltpu.VMEM((EB,), jnp.int32),
            pltpu.VMEM((B,), jnp.int32),
            pltpu.VMEM((B,), jnp.int32),
            pltpu.VMEM((B, D), jnp.float32),
            pltpu.VMEM_SHARED((R + PAD, D), jnp.float32),
        ],
    )
    def seg_kernel(x_hbm, src_hbm, dst_hbm, zacc_hbm, rows_out,
                   src_v, dst_v, idx_v, gidx_v, rows_v, acc_sp):
        c = lax.axis_index("c")
        s = lax.axis_index("s")
        ebase = pl.multiple_of(c * ep + s * te, B)

        for chunk in range(NCHUNK):
            lo = chunk * R
            zoff = pl.multiple_of(s * zrows, 8)
            pltpu.sync_copy(zacc_hbm.at[pl.ds(zoff, zrows)],
                            acc_sp.at[pl.ds(zoff, zrows)])
            plsc.subcore_barrier()

            @pl.loop(0, nstage)
            def _(st):
                soff = pl.multiple_of(ebase + st * EB, B)
                pltpu.sync_copy(src_hbm.at[pl.ds(soff, EB)], src_v)
                pltpu.sync_copy(dst_hbm.at[pl.ds(soff, EB)], dst_v)
                for j in range(EB // B):
                    off = j * B
                    for k in range(B // LANES):
                        d = dst_v[pl.ds(off + k * LANES, LANES)]
                        rel = d - lo
                        ok = (rel >= 0) & (rel < R)
                        idx_v[pl.ds(k * LANES, LANES)] = jnp.where(ok, rel, R)
                        gidx_v[pl.ds(k * LANES, LANES)] = (
                            src_v[pl.ds(off + k * LANES, LANES)])
                    pltpu.sync_copy(x_hbm.at[gidx_v], rows_v)
                    pltpu.sync_copy(rows_v, acc_sp.at[idx_v], add=True)

            plsc.subcore_barrier()
            obase = pl.multiple_of(c * OUTR + lo + s * orows, 8)
            roff = pl.multiple_of(s * orows, 8)
            pltpu.sync_copy(acc_sp.at[pl.ds(roff, orows)],
                            rows_out.at[pl.ds(obase, orows)])
            plsc.subcore_barrier()

    return seg_kernel(xcat, src_cat, dst_cat, zeros_acc)


def _sc_degree(dst_cat, ep):
    te = ep // NS
    nb = te // B
    zrows = DEGR // NS

    mesh = plsc.VectorSubcoreMesh(core_axis_name="c", subcore_axis_name="s",
                                  num_cores=NC, num_subcores=NS)

    @functools.partial(
        pl.kernel,
        mesh=mesh,
        out_type=jax.ShapeDtypeStruct((NC * DEGR,), jnp.float32),
        scratch_types=[
            pltpu.VMEM((te,), jnp.int32),
            pltpu.VMEM((B,), jnp.int32),
            pltpu.VMEM((B,), jnp.float32),
            pltpu.VMEM((zrows,), jnp.float32),
            pltpu.VMEM_SHARED((DEGR,), jnp.float32),
        ],
    )
    def deg_kernel(dst_hbm, deg_out, dst_v, idx_v, ones_v, st_v, deg_sp):
        c = lax.axis_index("c")
        s = lax.axis_index("s")
        ebase = pl.multiple_of(c * ep + s * te, B)
        for k in range(B // LANES):
            ones_v[pl.ds(k * LANES, LANES)] = jnp.full((LANES,), 1.0,
                                                       jnp.float32)

        @pl.loop(0, zrows // LANES)
        def _(i):
            st_v[pl.ds(i * LANES, LANES)] = jnp.zeros((LANES,), jnp.float32)

        pltpu.sync_copy(dst_hbm.at[pl.ds(ebase, te)], dst_v)
        zoff = pl.multiple_of(s * zrows, 8)
        pltpu.sync_copy(st_v, deg_sp.at[pl.ds(zoff, zrows)])
        plsc.subcore_barrier()

        @pl.loop(0, nb)
        def _(j):
            off = pl.multiple_of(j * B, B)
            for k in range(B // LANES):
                d = dst_v[pl.ds(off + k * LANES, LANES)]
                idx_v[pl.ds(k * LANES, LANES)] = jnp.where(d >= 0, d, N)
            pltpu.sync_copy(ones_v, deg_sp.at[idx_v], add=True)

        plsc.subcore_barrier()
        obase = pl.multiple_of(c * DEGR + s * zrows, 8)
        pltpu.sync_copy(deg_sp.at[pl.ds(zoff, zrows)], st_v)
        pltpu.sync_copy(st_v, deg_out.at[pl.ds(obase, zrows)])

    return deg_kernel(dst_cat)


def _tc_body(agg, deg, x, Wl, Wr, blb, g, b, o):
    mean = agg[0] / jnp.maximum(deg[0], 1.0)
    h = lax.dot_general(mean, Wl[0], (((1,), (1,)), ((), ())),
                        preferred_element_type=jnp.float32)
    h = h + blb[0]
    h = h + lax.dot_general(x[0], Wr[0], (((1,), (1,)), ((), ())),
                            preferred_element_type=jnp.float32)
    nrm = jnp.sqrt(jnp.sum(h * h, axis=1, keepdims=True))
    h = h / jnp.maximum(nrm, 1e-12)
    mu = jnp.mean(h, axis=1, keepdims=True)
    var = jnp.mean((h - mu) ** 2, axis=1, keepdims=True)
    hn = (h - mu) / jnp.sqrt(var + 1e-5) * g[0] + b[0]
    o[0] = jnp.maximum(hn, 0.0)


def _tc_dense(agg_st, deg_st, x_st, Wl_st, Wr_st, bl_st, g_st, b_st):
    blk = 1000
    nblk = N // blk
    row_spec = pl.BlockSpec((1, blk, D), lambda r, i: (r, i, 0))
    w_spec = pl.BlockSpec((1, D, D), lambda r, i: (r, 0, 0))
    v_spec = pl.BlockSpec((1, 1, D), lambda r, i: (r, 0, 0))
    return pl.pallas_call(
        _tc_body,
        grid=(2, nblk),
        in_specs=[row_spec, row_spec, row_spec, w_spec, w_spec,
                  v_spec, v_spec, v_spec],
        out_specs=row_spec,
        out_shape=jax.ShapeDtypeStruct((2, N, D), jnp.float32),
    )(agg_st, deg_st, x_st, Wl_st, Wr_st, bl_st, g_st, b_st)


def kernel(x_user, x_item, ei_rates, ei_rev, Wl_rates, bl_rates, Wr_rates,
           Wl_rev, bl_rev, Wr_rev, g_user, b_user, g_item, b_item):
    E = ei_rates.shape[1]
    ep = -(-E // (NS * B)) * (NS * B)

    xcat = jnp.concatenate([x_user, x_item], axis=0)

    def pad_rel(ei, src_off):
        src = jnp.full((ep,), src_off, jnp.int32).at[:E].set(ei[0] + src_off)
        dst = jnp.full((ep,), -1, jnp.int32).at[:E].set(ei[1])
        return src, dst

    src_rates, dst_rates = pad_rel(ei_rates, 0)
    src_rev, dst_rev = pad_rel(ei_rev, N)
    src_cat = jnp.concatenate([src_rates, src_rev])
    dst_cat = jnp.concatenate([dst_rates, dst_rev])

    rows = _sc_segment_sum(xcat, src_cat, dst_cat, ep)
    deg = _sc_degree(dst_cat, ep)

    agg_item, agg_user = rows[:N], rows[OUTR:OUTR + N]
    deg_item, deg_user = deg[:N], deg[DEGR:DEGR + N]

    agg_st = jnp.stack([agg_user, agg_item])
    deg_st = jnp.broadcast_to(jnp.stack([deg_user, deg_item])[:, :, None],
                              (2, N, D))
    x_st = jnp.stack([x_user, x_item])
    Wl_st = jnp.stack([Wl_rev, Wl_rates])
    Wr_st = jnp.stack([Wr_rev, Wr_rates])
    bl_st = jnp.stack([bl_rev, bl_rates]).reshape(2, 1, D)
    g_st = jnp.stack([g_user, g_item]).reshape(2, 1, D)
    b_st = jnp.stack([b_user, b_item]).reshape(2, 1, D)

    return _tc_dense(agg_st, deg_st, x_st, Wl_st, Wr_st, bl_st, g_st, b_st)


# speedup vs baseline: 2.0749x; 1.0801x over previous
import functools

import jax
import jax.numpy as jnp
from jax import lax
from jax.experimental import pallas as pl
from jax.experimental.pallas import tpu as pltpu
from jax.experimental.pallas import tpu_sc as plsc

N = 50000
D = 128

NC = 2
NS = 16
LANES = 16

B = 64
R = 12800
PAD = 128
NCHUNK = -(-N // R)
OUTR = NCHUNK * R


EB = 896
DEGR = 50176


def _sc_segment_sum(xcat, src_cat, dst_cat, ep):
    te = ep // NS
    nstage = te // EB
    zrows = (R + PAD) // NS
    orows = R // NS

    zeros_acc = jnp.zeros((R + PAD, D), jnp.float32)

    mesh = plsc.VectorSubcoreMesh(core_axis_name="c", subcore_axis_name="s",
                                  num_cores=NC, num_subcores=NS)

    @functools.partial(
        pl.kernel,
        mesh=mesh,
        out_type=jax.ShapeDtypeStruct((NC * OUTR, D), jnp.float32),
        scratch_types=[
            pltpu.VMEM((EB,), jnp.int32),
            pltpu.VMEM((EB,), jnp.int32),
            pltpu.VMEM((B,), jnp.int32),
            pltpu.VMEM((B,), jnp.int32),
            pltpu.VMEM((B,), jnp.int32),
            pltpu.VMEM((B,), jnp.int32),
            pltpu.VMEM((B, D), jnp.float32),
            pltpu.VMEM((B, D), jnp.float32),
            pltpu.SemaphoreType.DMA,
            pltpu.SemaphoreType.DMA,
            pltpu.VMEM_SHARED((R + PAD, D), jnp.float32),
        ],
    )
    def seg_kernel(x_hbm, src_hbm, dst_hbm, zacc_hbm, rows_out,
                   src_v, dst_v, idx_v0, idx_v1, gidx_v0, gidx_v1,
                   rows_v0, rows_v1, sem0, sem1, acc_sp):
        c = lax.axis_index("c")
        s = lax.axis_index("s")
        ebase = pl.multiple_of(c * ep + s * te, B)
        idx_v = (idx_v0, idx_v1)
        gidx_v = (gidx_v0, gidx_v1)
        rows_v = (rows_v0, rows_v1)
        sems = (sem0, sem1)
        nblk = EB // B

        for chunk in range(NCHUNK):
            lo = chunk * R
            zoff = pl.multiple_of(s * zrows, 8)
            pltpu.sync_copy(zacc_hbm.at[pl.ds(zoff, zrows)],
                            acc_sp.at[pl.ds(zoff, zrows)])
            plsc.subcore_barrier()

            @pl.loop(0, nstage)
            def _(st):
                soff = pl.multiple_of(ebase + st * EB, B)
                pltpu.sync_copy(src_hbm.at[pl.ds(soff, EB)], src_v)
                pltpu.sync_copy(dst_hbm.at[pl.ds(soff, EB)], dst_v)
                descs = [None, None]
                for j in range(nblk):
                    p = j & 1
                    off = j * B
                    for k in range(B // LANES):
                        d = dst_v[pl.ds(off + k * LANES, LANES)]
                        rel = d - lo
                        ok = (rel >= 0) & (rel < R)
                        idx_v[p][pl.ds(k * LANES, LANES)] = jnp.where(
                            ok, rel, R)
                        gidx_v[p][pl.ds(k * LANES, LANES)] = (
                            src_v[pl.ds(off + k * LANES, LANES)])
                    descs[p] = pltpu.async_copy(x_hbm.at[gidx_v[p]],
                                                rows_v[p], sems[p])
                    if j >= 1:
                        q = 1 - p
                        descs[q].wait()
                        pltpu.sync_copy(rows_v[q], acc_sp.at[idx_v[q]],
                                        add=True)
                q = (nblk - 1) & 1
                descs[q].wait()
                pltpu.sync_copy(rows_v[q], acc_sp.at[idx_v[q]], add=True)

            plsc.subcore_barrier()
            obase = pl.multiple_of(c * OUTR + lo + s * orows, 8)
            roff = pl.multiple_of(s * orows, 8)
            pltpu.sync_copy(acc_sp.at[pl.ds(roff, orows)],
                            rows_out.at[pl.ds(obase, orows)])
            plsc.subcore_barrier()

    return seg_kernel(xcat, src_cat, dst_cat, zeros_acc)


def _sc_degree(dst_cat, ep):
    te = ep // NS
    nb = te // B
    zrows = DEGR // NS

    mesh = plsc.VectorSubcoreMesh(core_axis_name="c", subcore_axis_name="s",
                                  num_cores=NC, num_subcores=NS)

    @functools.partial(
        pl.kernel,
        mesh=mesh,
        out_type=jax.ShapeDtypeStruct((NC * DEGR,), jnp.float32),
        scratch_types=[
            pltpu.VMEM((te,), jnp.int32),
            pltpu.VMEM((B,), jnp.int32),
            pltpu.VMEM((B,), jnp.float32),
            pltpu.VMEM((zrows,), jnp.float32),
            pltpu.VMEM_SHARED((DEGR,), jnp.float32),
        ],
    )
    def deg_kernel(dst_hbm, deg_out, dst_v, idx_v, ones_v, st_v, deg_sp):
        c = lax.axis_index("c")
        s = lax.axis_index("s")
        ebase = pl.multiple_of(c * ep + s * te, B)
        for k in range(B // LANES):
            ones_v[pl.ds(k * LANES, LANES)] = jnp.full((LANES,), 1.0,
                                                       jnp.float32)

        @pl.loop(0, zrows // LANES)
        def _(i):
            st_v[pl.ds(i * LANES, LANES)] = jnp.zeros((LANES,), jnp.float32)

        pltpu.sync_copy(dst_hbm.at[pl.ds(ebase, te)], dst_v)
        zoff = pl.multiple_of(s * zrows, 8)
        pltpu.sync_copy(st_v, deg_sp.at[pl.ds(zoff, zrows)])
        plsc.subcore_barrier()

        @pl.loop(0, nb)
        def _(j):
            off = pl.multiple_of(j * B, B)
            for k in range(B // LANES):
                d = dst_v[pl.ds(off + k * LANES, LANES)]
                idx_v[pl.ds(k * LANES, LANES)] = jnp.where(d >= 0, d, N)
            pltpu.sync_copy(ones_v, deg_sp.at[idx_v], add=True)

        plsc.subcore_barrier()
        obase = pl.multiple_of(c * DEGR + s * zrows, 8)
        pltpu.sync_copy(deg_sp.at[pl.ds(zoff, zrows)], st_v)
        pltpu.sync_copy(st_v, deg_out.at[pl.ds(obase, zrows)])

    return deg_kernel(dst_cat)


def _tc_body(agg, deg, x, Wl, Wr, blb, g, b, o):
    mean = agg[0] / jnp.maximum(deg[0], 1.0)
    h = lax.dot_general(mean, Wl[0], (((1,), (1,)), ((), ())),
                        preferred_element_type=jnp.float32)
    h = h + blb[0]
    h = h + lax.dot_general(x[0], Wr[0], (((1,), (1,)), ((), ())),
                            preferred_element_type=jnp.float32)
    nrm = jnp.sqrt(jnp.sum(h * h, axis=1, keepdims=True))
    h = h / jnp.maximum(nrm, 1e-12)
    mu = jnp.mean(h, axis=1, keepdims=True)
    var = jnp.mean((h - mu) ** 2, axis=1, keepdims=True)
    hn = (h - mu) / jnp.sqrt(var + 1e-5) * g[0] + b[0]
    o[0] = jnp.maximum(hn, 0.0)


def _tc_dense(agg_st, deg_st, x_st, Wl_st, Wr_st, bl_st, g_st, b_st):
    blk = 1000
    nblk = N // blk
    row_spec = pl.BlockSpec((1, blk, D), lambda r, i: (r, i, 0))
    w_spec = pl.BlockSpec((1, D, D), lambda r, i: (r, 0, 0))
    v_spec = pl.BlockSpec((1, 1, D), lambda r, i: (r, 0, 0))
    return pl.pallas_call(
        _tc_body,
        grid=(2, nblk),
        in_specs=[row_spec, row_spec, row_spec, w_spec, w_spec,
                  v_spec, v_spec, v_spec],
        out_specs=row_spec,
        out_shape=jax.ShapeDtypeStruct((2, N, D), jnp.float32),
    )(agg_st, deg_st, x_st, Wl_st, Wr_st, bl_st, g_st, b_st)


def kernel(x_user, x_item, ei_rates, ei_rev, Wl_rates, bl_rates, Wr_rates,
           Wl_rev, bl_rev, Wr_rev, g_user, b_user, g_item, b_item):
    E = ei_rates.shape[1]
    ep = -(-E // (NS * EB)) * (NS * EB)

    xcat = jnp.concatenate([x_user, x_item], axis=0)

    def pad_rel(ei, src_off):
        src = jnp.full((ep,), src_off, jnp.int32).at[:E].set(ei[0] + src_off)
        dst = jnp.full((ep,), -1, jnp.int32).at[:E].set(ei[1])
        return src, dst

    src_rates, dst_rates = pad_rel(ei_rates, 0)
    src_rev, dst_rev = pad_rel(ei_rev, N)
    src_cat = jnp.concatenate([src_rates, src_rev])
    dst_cat = jnp.concatenate([dst_rates, dst_rev])

    rows = _sc_segment_sum(xcat, src_cat, dst_cat, ep)
    deg = _sc_degree(dst_cat, ep)

    agg_item, agg_user = rows[:N], rows[OUTR:OUTR + N]
    deg_item, deg_user = deg[:N], deg[DEGR:DEGR + N]

    agg_st = jnp.stack([agg_user, agg_item])
    deg_st = jnp.broadcast_to(jnp.stack([deg_user, deg_item])[:, :, None],
                              (2, N, D))
    x_st = jnp.stack([x_user, x_item])
    Wl_st = jnp.stack([Wl_rev, Wl_rates])
    Wr_st = jnp.stack([Wr_rev, Wr_rates])
    bl_st = jnp.stack([bl_rev, bl_rates]).reshape(2, 1, D)
    g_st = jnp.stack([g_user, g_item]).reshape(2, 1, D)
    b_st = jnp.stack([b_user, b_item]).reshape(2, 1, D)

    return _tc_dense(agg_st, deg_st, x_st, Wl_st, Wr_st, bl_st, g_st, b_st)


# speedup vs baseline: 2.1046x; 1.0143x over previous
import functools

import jax
import jax.numpy as jnp
from jax import lax
from jax.experimental import pallas as pl
from jax.experimental.pallas import tpu as pltpu
from jax.experimental.pallas import tpu_sc as plsc

N = 50000
D = 128

NC = 2
NS = 16
LANES = 16

B = 64
R = 12800
PAD = 128
NCHUNK = -(-N // R)
OUTR = NCHUNK * R


EB = 2240
DEGR = 50176


def _sc_segment_sum(xcat, src_cat, dst_cat, ep):
    te = ep // NS
    nstage = te // EB
    zrows = (R + PAD) // NS
    orows = R // NS

    zeros_acc = jnp.zeros((R + PAD, D), jnp.float32)

    mesh = plsc.VectorSubcoreMesh(core_axis_name="c", subcore_axis_name="s",
                                  num_cores=NC, num_subcores=NS)

    @functools.partial(
        pl.kernel,
        mesh=mesh,
        out_type=jax.ShapeDtypeStruct((NC * OUTR, D), jnp.float32),
        scratch_types=[
            pltpu.VMEM((EB,), jnp.int32),
            pltpu.VMEM((EB,), jnp.int32),
            pltpu.VMEM((B,), jnp.int32),
            pltpu.VMEM((B,), jnp.int32),
            pltpu.VMEM((B,), jnp.int32),
            pltpu.VMEM((B,), jnp.int32),
            pltpu.VMEM((B, D), jnp.float32),
            pltpu.VMEM((B, D), jnp.float32),
            pltpu.SemaphoreType.DMA,
            pltpu.SemaphoreType.DMA,
            pltpu.VMEM_SHARED((R + PAD, D), jnp.float32),
        ],
    )
    def seg_kernel(x_hbm, src_hbm, dst_hbm, zacc_hbm, rows_out,
                   src_v, dst_v, idx_v0, idx_v1, gidx_v0, gidx_v1,
                   rows_v0, rows_v1, sem0, sem1, acc_sp):
        c = lax.axis_index("c")
        s = lax.axis_index("s")
        ebase = pl.multiple_of(c * ep + s * te, B)
        idx_v = (idx_v0, idx_v1)
        gidx_v = (gidx_v0, gidx_v1)
        rows_v = (rows_v0, rows_v1)
        sems = (sem0, sem1)

        for chunk in range(NCHUNK):
            lo = chunk * R
            zoff = pl.multiple_of(s * zrows, 8)
            pltpu.sync_copy(zacc_hbm.at[pl.ds(zoff, zrows)],
                            acc_sp.at[pl.ds(zoff, zrows)])
            plsc.subcore_barrier()

            @pl.loop(0, nstage)
            def _(st):
                soff = pl.multiple_of(ebase + st * EB, B)
                pltpu.sync_copy(src_hbm.at[pl.ds(soff, EB)], src_v)
                pltpu.sync_copy(dst_hbm.at[pl.ds(soff, EB)], dst_v)
                descs = [None, None]
                for j in range(EB // B):
                    p = j & 1
                    off = j * B
                    for k in range(B // LANES):
                        d = dst_v[pl.ds(off + k * LANES, LANES)]
                        rel = d - lo
                        ok = (rel >= 0) & (rel < R)
                        idx_v[p][pl.ds(k * LANES, LANES)] = jnp.where(
                            ok, rel, R)
                        gidx_v[p][pl.ds(k * LANES, LANES)] = (
                            src_v[pl.ds(off + k * LANES, LANES)])
                    descs[p] = pltpu.async_copy(x_hbm.at[gidx_v[p]],
                                                rows_v[p], sems[p])
                    if j >= 1:
                        q = 1 - p
                        descs[q].wait()
                        pltpu.sync_copy(rows_v[q], acc_sp.at[idx_v[q]],
                                        add=True)
                q = (EB // B - 1) & 1
                descs[q].wait()
                pltpu.sync_copy(rows_v[q], acc_sp.at[idx_v[q]], add=True)

            plsc.subcore_barrier()
            obase = pl.multiple_of(c * OUTR + lo + s * orows, 8)
            roff = pl.multiple_of(s * orows, 8)
            pltpu.sync_copy(acc_sp.at[pl.ds(roff, orows)],
                            rows_out.at[pl.ds(obase, orows)])
            plsc.subcore_barrier()

    return seg_kernel(xcat, src_cat, dst_cat, zeros_acc)


def _sc_degree(dst_cat, ep):
    te = ep // NS
    nb = te // B
    zrows = DEGR // NS

    mesh = plsc.VectorSubcoreMesh(core_axis_name="c", subcore_axis_name="s",
                                  num_cores=NC, num_subcores=NS)

    @functools.partial(
        pl.kernel,
        mesh=mesh,
        out_type=jax.ShapeDtypeStruct((NC * DEGR,), jnp.float32),
        scratch_types=[
            pltpu.VMEM((te,), jnp.int32),
            pltpu.VMEM((B,), jnp.int32),
            pltpu.VMEM((B,), jnp.float32),
            pltpu.VMEM((zrows,), jnp.float32),
            pltpu.VMEM_SHARED((DEGR,), jnp.float32),
        ],
    )
    def deg_kernel(dst_hbm, deg_out, dst_v, idx_v, ones_v, st_v, deg_sp):
        c = lax.axis_index("c")
        s = lax.axis_index("s")
        ebase = pl.multiple_of(c * ep + s * te, B)
        for k in range(B // LANES):
            ones_v[pl.ds(k * LANES, LANES)] = jnp.full((LANES,), 1.0,
                                                       jnp.float32)

        @pl.loop(0, zrows // LANES)
        def _(i):
            st_v[pl.ds(i * LANES, LANES)] = jnp.zeros((LANES,), jnp.float32)

        pltpu.sync_copy(dst_hbm.at[pl.ds(ebase, te)], dst_v)
        zoff = pl.multiple_of(s * zrows, 8)
        pltpu.sync_copy(st_v, deg_sp.at[pl.ds(zoff, zrows)])
        plsc.subcore_barrier()

        @pl.loop(0, nb)
        def _(j):
            off = pl.multiple_of(j * B, B)
            for k in range(B // LANES):
                d = dst_v[pl.ds(off + k * LANES, LANES)]
                idx_v[pl.ds(k * LANES, LANES)] = jnp.where(d >= 0, d, N)
            pltpu.sync_copy(ones_v, deg_sp.at[idx_v], add=True)

        plsc.subcore_barrier()
        obase = pl.multiple_of(c * DEGR + s * zrows, 8)
        pltpu.sync_copy(deg_sp.at[pl.ds(zoff, zrows)], st_v)
        pltpu.sync_copy(st_v, deg_out.at[pl.ds(obase, zrows)])

    return deg_kernel(dst_cat)


def _tc_body(agg, deg, x, Wl, Wr, blb, g, b, o):
    mean = agg[0] / jnp.maximum(deg[0], 1.0)
    h = lax.dot_general(mean, Wl[0], (((1,), (1,)), ((), ())),
                        preferred_element_type=jnp.float32)
    h = h + blb[0]
    h = h + lax.dot_general(x[0], Wr[0], (((1,), (1,)), ((), ())),
                            preferred_element_type=jnp.float32)
    nrm = jnp.sqrt(jnp.sum(h * h, axis=1, keepdims=True))
    h = h / jnp.maximum(nrm, 1e-12)
    mu = jnp.mean(h, axis=1, keepdims=True)
    var = jnp.mean((h - mu) ** 2, axis=1, keepdims=True)
    hn = (h - mu) / jnp.sqrt(var + 1e-5) * g[0] + b[0]
    o[0] = jnp.maximum(hn, 0.0)


def _tc_dense(agg_st, deg_st, x_st, Wl_st, Wr_st, bl_st, g_st, b_st):
    blk = 1000
    nblk = N // blk
    row_spec = pl.BlockSpec((1, blk, D), lambda r, i: (r, i, 0))
    w_spec = pl.BlockSpec((1, D, D), lambda r, i: (r, 0, 0))
    v_spec = pl.BlockSpec((1, 1, D), lambda r, i: (r, 0, 0))
    return pl.pallas_call(
        _tc_body,
        grid=(2, nblk),
        in_specs=[row_spec, row_spec, row_spec, w_spec, w_spec,
                  v_spec, v_spec, v_spec],
        out_specs=row_spec,
        out_shape=jax.ShapeDtypeStruct((2, N, D), jnp.float32),
    )(agg_st, deg_st, x_st, Wl_st, Wr_st, bl_st, g_st, b_st)


def kernel(x_user, x_item, ei_rates, ei_rev, Wl_rates, bl_rates, Wr_rates,
           Wl_rev, bl_rev, Wr_rev, g_user, b_user, g_item, b_item):
    E = ei_rates.shape[1]
    ep = -(-E // (NS * EB)) * (NS * EB)

    xcat = jnp.concatenate([x_user, x_item], axis=0)

    def pad_rel(ei, src_off):
        src = jnp.full((ep,), src_off, jnp.int32).at[:E].set(ei[0] + src_off)
        dst = jnp.full((ep,), -1, jnp.int32).at[:E].set(ei[1])
        return src, dst

    src_rates, dst_rates = pad_rel(ei_rates, 0)
    src_rev, dst_rev = pad_rel(ei_rev, N)
    src_cat = jnp.concatenate([src_rates, src_rev])
    dst_cat = jnp.concatenate([dst_rates, dst_rev])

    rows = _sc_segment_sum(xcat, src_cat, dst_cat, ep)
    deg = _sc_degree(dst_cat, ep)

    agg_item, agg_user = rows[:N], rows[OUTR:OUTR + N]
    deg_item, deg_user = deg[:N], deg[DEGR:DEGR + N]

    agg_st = jnp.stack([agg_user, agg_item])
    deg_st = jnp.broadcast_to(jnp.stack([deg_user, deg_item])[:, :, None],
                              (2, N, D))
    x_st = jnp.stack([x_user, x_item])
    Wl_st = jnp.stack([Wl_rev, Wl_rates])
    Wr_st = jnp.stack([Wr_rev, Wr_rates])
    bl_st = jnp.stack([bl_rev, bl_rates]).reshape(2, 1, D)
    g_st = jnp.stack([g_user, g_item]).reshape(2, 1, D)
    b_st = jnp.stack([b_user, b_item]).reshape(2, 1, D)

    return _tc_dense(agg_st, deg_st, x_st, Wl_st, Wr_st, bl_st, g_st, b_st)
